# baseline (speedup 1.0000x reference)
import jax
import jax.numpy as jnp
from jax.experimental import pallas as pl
from jax.experimental.pallas import tpu as pltpu

N = 10000
D = 128
BLOCK = 2000

_DN = (((1,), (1,)), ((), ()))


def _mlp3_kernel(x_ref, w0_ref, w1_ref, w2_ref, b0_ref, b1_ref, b2_ref,
                 out_ref):
    x = x_ref[...].astype(jnp.bfloat16)
    h = jax.lax.dot_general(x, w0_ref[...].astype(jnp.bfloat16), _DN,
                            preferred_element_type=jnp.float32)
    h = jnp.maximum(h + b0_ref[...], 0.0).astype(jnp.bfloat16)
    h = jax.lax.dot_general(h, w1_ref[...].astype(jnp.bfloat16), _DN,
                            preferred_element_type=jnp.float32)
    h = jnp.maximum(h + b1_ref[...], 0.0).astype(jnp.bfloat16)
    h = jax.lax.dot_general(h, w2_ref[...].astype(jnp.bfloat16), _DN,
                            preferred_element_type=jnp.float32)
    out_ref[...] = h + b2_ref[...]


def kernel(x, edge_index, edge_weight, W0, b0, W1, b1, W2, b2):
    grid = (N // BLOCK,)
    full = pl.BlockSpec((D, D), lambda i: (0, 0))
    brow = pl.BlockSpec((1, D), lambda i: (0, 0))
    out = pl.pallas_call(
        _mlp3_kernel,
        grid=grid,
        in_specs=[
            pl.BlockSpec((BLOCK, D), lambda i: (i, 0)),
            full, full, full,
            brow, brow, brow,
        ],
        out_specs=pl.BlockSpec((BLOCK, D), lambda i: (i, 0)),
        out_shape=jax.ShapeDtypeStruct((N, D), jnp.float32),
        compiler_params=pltpu.CompilerParams(
            dimension_semantics=("parallel",),
        ),
    )(x, W0, W1, W2,
      b0.reshape(1, D), b1.reshape(1, D), b2.reshape(1, D))
    return out


# speedup vs baseline: 1.1376x; 1.1376x over previous
import jax
import jax.numpy as jnp
from jax.experimental import pallas as pl
from jax.experimental.pallas import tpu as pltpu

N = 10000
D = 128
BLOCK = 2000

_DN = (((1,), (1,)), ((), ()))


def _mlp3_kernel(x_ref, w0_ref, w1_ref, w2_ref, b0_ref, b1_ref, b2_ref,
                 out_ref):
    x = x_ref[...]
    h = jax.lax.dot_general(x, w0_ref[...], _DN,
                            preferred_element_type=jnp.float32)
    h = jnp.maximum(h + b0_ref[...], 0.0)
    h = jax.lax.dot_general(h, w1_ref[...], _DN,
                            preferred_element_type=jnp.float32)
    h = jnp.maximum(h + b1_ref[...], 0.0)
    h = jax.lax.dot_general(h, w2_ref[...], _DN,
                            preferred_element_type=jnp.float32)
    out_ref[...] = h + b2_ref[...]


def kernel(x, edge_index, edge_weight, W0, b0, W1, b1, W2, b2):
    grid = (N // BLOCK,)
    full = pl.BlockSpec((D, D), lambda i: (0, 0))
    brow = pl.BlockSpec((1, D), lambda i: (0, 0))
    out = pl.pallas_call(
        _mlp3_kernel,
        grid=grid,
        in_specs=[
            pl.BlockSpec((BLOCK, D), lambda i: (i, 0)),
            full, full, full,
            brow, brow, brow,
        ],
        out_specs=pl.BlockSpec((BLOCK, D), lambda i: (i, 0)),
        out_shape=jax.ShapeDtypeStruct((N, D), jnp.float32),
        compiler_params=pltpu.CompilerParams(
            dimension_semantics=("parallel",),
        ),
    )(x, W0, W1, W2,
      b0.reshape(1, D), b1.reshape(1, D), b2.reshape(1, D))
    return out


# speedup vs baseline: 1.1379x; 1.0002x over previous
import jax
import jax.numpy as jnp
from jax.experimental import pallas as pl
from jax.experimental.pallas import tpu as pltpu

N = 10000
D = 128
BLOCK = 2000

_DN = (((1,), (1,)), ((), ()))


def _mlp3_kernel(x_ref, w0_ref, w1_ref, w2_ref, b0_ref, b1_ref, b2_ref,
                 out_ref):
    x = x_ref[...]
    h = jax.lax.dot_general(x, w0_ref[...], _DN,
                            preferred_element_type=jnp.float32)
    h = jnp.maximum(h + b0_ref[...], 0.0)
    h = jax.lax.dot_general(h, w1_ref[...], _DN,
                            preferred_element_type=jnp.float32)
    h = jnp.maximum(h + b1_ref[...], 0.0)
    h = jax.lax.dot_general(h, w2_ref[...], _DN,
                            preferred_element_type=jnp.float32)
    out_ref[...] = h + b2_ref[...]


def kernel(x, edge_index, edge_weight, W0, b0, W1, b1, W2, b2):
    grid = (N // BLOCK,)
    full = pl.BlockSpec((D, D), lambda i: (0, 0))
    brow = pl.BlockSpec((1, D), lambda i: (0, 0))
    out = pl.pallas_call(
        _mlp3_kernel,
        grid=grid,
        in_specs=[
            pl.BlockSpec((BLOCK, D), lambda i: (i, 0)),
            full, full, full,
            brow, brow, brow,
        ],
        out_specs=pl.BlockSpec((BLOCK, D), lambda i: (i, 0)),
        out_shape=jax.ShapeDtypeStruct((N, D), jnp.float32),
        compiler_params=pltpu.CompilerParams(
            dimension_semantics=("arbitrary",),
        ),
    )(x, W0, W1, W2,
      b0.reshape(1, D), b1.reshape(1, D), b2.reshape(1, D))
    return out
